# baseline (speedup 1.0000x reference)
import functools

import jax
import jax.numpy as jnp
from jax import lax
from jax.experimental import pallas as pl
from jax.experimental.pallas import tpu as pltpu
from jax.experimental.pallas import tpu_sc as plsc

N, M, T, D = 1024, 512, 50, 2
ROW = T * D
PAD = 128
NLINES = N * M * ROW // PAD
NS = 16
BPW = N // NS
LANES = 16
_INV_N = 1.0 / N


def _prep_body(cls_ref, e_ref, at_ref, ids_ref, k0_ref, d_ref, closs_ref):
    d0 = e_ref[:, 0:1] - at_ref[0:1, :]
    d1 = e_ref[:, 1:2] - at_ref[1:2, :]
    dists = d0 * d0 + d1 * d1
    mind = jnp.min(dists, axis=1, keepdims=True)
    col = lax.broadcasted_iota(jnp.int32, (N, M), 1)
    ids = jnp.min(jnp.where(dists <= mind, col, M), axis=1, keepdims=True)

    cls = cls_ref[...]
    sel = jnp.sum(jnp.where(col == ids, cls, 0.0), axis=1, keepdims=True)
    mx = jnp.max(cls, axis=1, keepdims=True)
    lse = jnp.log(jnp.sum(jnp.exp(cls - mx), axis=1, keepdims=True)) + mx
    closs_ref[0, 0] = jnp.sum(lse - sel) * _INV_N

    ids_ref[...] = ids
    rows = lax.broadcasted_iota(jnp.int32, (N, 1), 0)
    w0 = (rows * M + ids) * ROW
    k0_ref[...] = lax.shift_right_logical(w0, 7)
    d_ref[...] = lax.bitwise_and(w0, PAD - 1)


def _sc_body(gid_hbm, ids_hbm, dv_hbm, tgt_hbm, msk_hbm, reg_hbm, anc_hbm,
             cls_hbm, out_hbm, gid_v, ids_v, d_v, reg_v, anc_v, tgt_v, msk_v,
             part_s, part_v, sum_v, cls_v, out_v, sem_r, sem_a):
    wid = lax.axis_index("s")
    base = wid * BPW
    pltpu.sync_copy(gid_hbm.at[pl.ds(2 * base, 2 * BPW)], gid_v)
    pltpu.sync_copy(ids_hbm.at[pl.ds(base, BPW)], ids_v)
    pltpu.sync_copy(dv_hbm.at[pl.ds(base, BPW)], d_v)
    cp_r = pltpu.async_copy(reg_hbm.at[gid_v], reg_v, sem_r)
    cp_a = pltpu.async_copy(anc_hbm.at[ids_v], anc_v, sem_a)
    pltpu.sync_copy(tgt_hbm.at[pl.ds(base, BPW)], tgt_v)
    pltpu.sync_copy(msk_hbm.at[pl.ds(base, BPW)], msk_v)
    cp_r.wait()
    cp_a.wait()

    lane = lax.iota(jnp.int32, LANES)

    def row_body(i, acc):
        db = plsc.load_gather(d_v, [jnp.full((LANES,), i, jnp.int32)])
        for c in range(7):
            off = c * LANES
            pos = db + (off + lane)
            rrow = lax.shift_right_logical(pos, 7) + 2 * i
            rcol = lax.bitwise_and(pos, PAD - 1)
            r = plsc.load_gather(reg_v, [rrow, rcol])
            a = anc_v[i, pl.ds(off, LANES)]
            t = tgt_v[i, pl.ds(off, LANES)]
            m = msk_v[i, pl.ds(off, LANES)]
            diff = (r + a) - t
            ad = jnp.abs(diff)
            sl1 = jnp.where(ad < 1.0, 0.5 * diff * diff, ad - 0.5)
            acc = acc + jnp.where(m != 0.0, sl1, 0.0)
        return acc

    acc = lax.fori_loop(0, BPW, row_body, jnp.zeros((LANES,), jnp.float32))
    sum_v[...] = acc
    pltpu.sync_copy(sum_v, part_s.at[wid])
    plsc.subcore_barrier()

    @pl.when(wid == 0)
    def _():
        pltpu.sync_copy(part_s, part_v)
        pltpu.sync_copy(cls_hbm, cls_v)

        def fold(i, acc):
            return acc + part_v[i, :]

        tot = lax.fori_loop(0, NS, fold, jnp.zeros((LANES,), jnp.float32))
        out_v[...] = cls_v[...] * (1.0 / LANES) + tot * _INV_N
        pltpu.sync_copy(out_v, out_hbm)


@jax.jit
def kernel(cls_out, reg_out, targets, anchors):
    e = targets[0, :, -1, :]
    a_t = anchors[:, -1, :].T
    ids2, k0, dvals, closs = pl.pallas_call(
        _prep_body,
        out_shape=[
            jax.ShapeDtypeStruct((N, 1), jnp.int32),
            jax.ShapeDtypeStruct((N, 1), jnp.int32),
            jax.ShapeDtypeStruct((N, 1), jnp.int32),
            jax.ShapeDtypeStruct((1, 1), jnp.float32),
        ],
        out_specs=[
            pl.BlockSpec(memory_space=pltpu.VMEM),
            pl.BlockSpec(memory_space=pltpu.VMEM),
            pl.BlockSpec(memory_space=pltpu.VMEM),
            pl.BlockSpec(memory_space=pltpu.SMEM),
        ],
    )(cls_out, e, a_t)

    reg128 = reg_out.reshape(NLINES, PAD)
    anc128 = jnp.pad(anchors.reshape(M, ROW), ((0, 0), (0, PAD - ROW)))
    tgt128 = jnp.pad(targets[0].reshape(N, ROW), ((0, 0), (0, PAD - ROW)))
    msk128 = jnp.pad(targets[1].reshape(N, ROW), ((0, 0), (0, PAD - ROW)))
    gidx = jnp.concatenate([k0, k0 + 1], axis=1).reshape(2 * N)
    cls16 = jnp.broadcast_to(closs.reshape(1), (LANES,))

    mesh = plsc.VectorSubcoreMesh(
        core_axis_name="c", subcore_axis_name="s", num_cores=1)
    sc = pl.kernel(
        _sc_body,
        out_type=jax.ShapeDtypeStruct((LANES,), jnp.float32),
        mesh=mesh,
        compiler_params=pltpu.CompilerParams(
            use_tc_tiling_on_sc=False, needs_layout_passes=False),
        scratch_types=[
            pltpu.VMEM((2 * BPW,), jnp.int32),
            pltpu.VMEM((BPW,), jnp.int32),
            pltpu.VMEM((BPW,), jnp.int32),
            pltpu.VMEM((2 * BPW, PAD), jnp.float32),
            pltpu.VMEM((BPW, PAD), jnp.float32),
            pltpu.VMEM((BPW, PAD), jnp.float32),
            pltpu.VMEM((BPW, PAD), jnp.float32),
            pltpu.VMEM_SHARED((NS, LANES), jnp.float32),
            pltpu.VMEM((NS, LANES), jnp.float32),
            pltpu.VMEM((LANES,), jnp.float32),
            pltpu.VMEM((LANES,), jnp.float32),
            pltpu.VMEM((LANES,), jnp.float32),
            pltpu.SemaphoreType.DMA,
            pltpu.SemaphoreType.DMA,
        ],
    )
    out16 = sc(gidx, ids2.reshape(N), dvals.reshape(N), tgt128, msk128,
               reg128, anc128, cls16)
    return jnp.sum(out16)


# speedup vs baseline: 829.1342x; 829.1342x over previous
import functools

import jax
import jax.numpy as jnp
from jax import lax
from jax.experimental import pallas as pl
from jax.experimental.pallas import tpu as pltpu
from jax.experimental.pallas import tpu_sc as plsc

N, M, T, D = 1024, 512, 50, 2
ROW = T * D
PAD = 128
NW = N * M * T * D
NS = 16
BPW = N // NS
LANES = 16
_INV_N = 1.0 / N


def _prep_body(cls_ref, e_ref, at_ref, tgt_ref, anc_ref, gi_ref, tm_ref,
               closs_ref):
    d0 = e_ref[:, 0:1] - at_ref[0:1, :]
    d1 = e_ref[:, 1:2] - at_ref[1:2, :]
    dists = d0 * d0 + d1 * d1
    mind = jnp.min(dists, axis=1, keepdims=True)
    col = lax.broadcasted_iota(jnp.int32, (N, M), 1)
    ids = jnp.min(jnp.where(dists <= mind, col, M), axis=1, keepdims=True)

    cls = cls_ref[...]
    onehot = col == ids
    sel = jnp.sum(jnp.where(onehot, cls, 0.0), axis=1, keepdims=True)
    mx = jnp.max(cls, axis=1, keepdims=True)
    lse = jnp.log(jnp.sum(jnp.exp(cls - mx), axis=1, keepdims=True)) + mx
    closs_ref[0, 0] = jnp.sum(lse - sel) * _INV_N

    rows = lax.broadcasted_iota(jnp.int32, (N, 1), 0)
    colp = lax.broadcasted_iota(jnp.int32, (N, PAD), 1)
    jj = jnp.minimum(colp, ROW - 1)
    t = lax.shift_right_logical(jj, 1)
    d = lax.bitwise_and(jj, 1)
    gi_ref[...] = (rows * (M * ROW) + t * (2 * M) + d * 128
                   + lax.shift_right_logical(ids, 7) * 256
                   + lax.bitwise_and(ids, 127))

    asel = jax.lax.dot_general(
        jnp.where(onehot, 1.0, 0.0), anc_ref[...],
        dimension_numbers=(((1,), (0,)), ((), ())),
        preferred_element_type=jnp.float32)
    tm_ref[...] = tgt_ref[...] - asel


def _sc_body(gi_hbm, tm_hbm, msk_hbm, reg_hbm, cls_hbm, out_hbm,
             gi_v, reg_v, tm_v, msk_v, part_s, part_v, sum_v, cls_v, out_v,
             sem_r):
    wid = lax.axis_index("s")
    base = wid * BPW
    pltpu.sync_copy(gi_hbm.at[pl.ds(base * PAD, BPW * PAD)], gi_v)
    cp_r = pltpu.async_copy(reg_hbm.at[gi_v], reg_v, sem_r)
    pltpu.sync_copy(tm_hbm.at[pl.ds(base, BPW)], tm_v)
    pltpu.sync_copy(msk_hbm.at[pl.ds(base, BPW)], msk_v)
    cp_r.wait()

    def row_body(i, acc):
        for c in range(7):
            off = c * LANES
            r = reg_v[pl.ds(i * PAD + off, LANES)]
            tm = tm_v[i, pl.ds(off, LANES)]
            m = msk_v[i, pl.ds(off, LANES)]
            diff = r - tm
            ad = jnp.abs(diff)
            sl1 = jnp.where(ad < 1.0, 0.5 * diff * diff, ad - 0.5)
            acc = acc + jnp.where(m != 0.0, sl1, 0.0)
        return acc

    acc = lax.fori_loop(0, BPW, row_body, jnp.zeros((LANES,), jnp.float32))
    sum_v[...] = acc
    pltpu.sync_copy(sum_v, part_s.at[wid])
    plsc.subcore_barrier()

    @pl.when(wid == 0)
    def _():
        pltpu.sync_copy(part_s, part_v)
        pltpu.sync_copy(cls_hbm, cls_v)

        def fold(i, acc):
            return acc + part_v[i, :]

        tot = lax.fori_loop(0, NS, fold, jnp.zeros((LANES,), jnp.float32))
        out_v[...] = cls_v[...] * (1.0 / LANES) + tot * _INV_N
        pltpu.sync_copy(out_v, out_hbm)


@jax.jit
def kernel(cls_out, reg_out, targets, anchors):
    e = targets[0, :, -1, :]
    a_t = anchors[:, -1, :].T
    tgt2p = jnp.pad(targets[0].reshape(N, ROW), ((0, 0), (0, PAD - ROW)))
    msk2p = jnp.pad(targets[1].reshape(N, ROW), ((0, 0), (0, PAD - ROW)))
    anc2p = jnp.pad(anchors.reshape(M, ROW), ((0, 0), (0, PAD - ROW)))

    gi, tm, closs = pl.pallas_call(
        _prep_body,
        out_shape=[
            jax.ShapeDtypeStruct((N, PAD), jnp.int32),
            jax.ShapeDtypeStruct((N, PAD), jnp.float32),
            jax.ShapeDtypeStruct((1, 1), jnp.float32),
        ],
        out_specs=[
            pl.BlockSpec(memory_space=pltpu.VMEM),
            pl.BlockSpec(memory_space=pltpu.VMEM),
            pl.BlockSpec(memory_space=pltpu.SMEM),
        ],
    )(cls_out, e, a_t, tgt2p, anc2p)

    regflat = (reg_out.reshape(N, 4, 128, T, D)
               .transpose(0, 3, 1, 4, 2).reshape(NW))
    cls16 = jnp.broadcast_to(closs.reshape(1), (LANES,))

    mesh = plsc.VectorSubcoreMesh(
        core_axis_name="c", subcore_axis_name="s", num_cores=1)
    sc = pl.kernel(
        _sc_body,
        out_type=jax.ShapeDtypeStruct((LANES,), jnp.float32),
        mesh=mesh,
        compiler_params=pltpu.CompilerParams(
            use_tc_tiling_on_sc=False, needs_layout_passes=False),
        scratch_types=[
            pltpu.VMEM((BPW * PAD,), jnp.int32),
            pltpu.VMEM((BPW * PAD,), jnp.float32),
            pltpu.VMEM((BPW, PAD), jnp.float32),
            pltpu.VMEM((BPW, PAD), jnp.float32),
            pltpu.VMEM_SHARED((NS, LANES), jnp.float32),
            pltpu.VMEM((NS, LANES), jnp.float32),
            pltpu.VMEM((LANES,), jnp.float32),
            pltpu.VMEM((LANES,), jnp.float32),
            pltpu.VMEM((LANES,), jnp.float32),
            pltpu.SemaphoreType.DMA,
        ],
    )
    out16 = sc(gi.reshape(N * PAD), tm, msk2p, regflat, cls16)
    return jnp.sum(out16)


# speedup vs baseline: 886.7255x; 1.0695x over previous
import functools

import jax
import jax.numpy as jnp
from jax import lax
from jax.experimental import pallas as pl
from jax.experimental.pallas import tpu as pltpu
from jax.experimental.pallas import tpu_sc as plsc

N, M, T, D = 1024, 512, 50, 2
ROW = T * D
PAD = 128
NW = N * M * T * D
NC = 2
NS = 16
NWK = NC * NS
BPW = N // NWK
LANES = 16
_INV_N = 1.0 / N


def _prep_body(cls_ref, e_ref, at_ref, tgt_ref, anc_ref, gi_ref, tm_ref,
               closs_ref):
    d0 = e_ref[:, 0:1] - at_ref[0:1, :]
    d1 = e_ref[:, 1:2] - at_ref[1:2, :]
    dists = d0 * d0 + d1 * d1
    mind = jnp.min(dists, axis=1, keepdims=True)
    col = lax.broadcasted_iota(jnp.int32, (N, M), 1)
    ids = jnp.min(jnp.where(dists <= mind, col, M), axis=1, keepdims=True)

    cls = cls_ref[...]
    onehot = col == ids
    sel = jnp.sum(jnp.where(onehot, cls, 0.0), axis=1, keepdims=True)
    mx = jnp.max(cls, axis=1, keepdims=True)
    lse = jnp.log(jnp.sum(jnp.exp(cls - mx), axis=1, keepdims=True)) + mx
    closs_ref[0, 0] = jnp.sum(lse - sel) * _INV_N

    rows = lax.broadcasted_iota(jnp.int32, (N, 1), 0)
    colp = lax.broadcasted_iota(jnp.int32, (N, PAD), 1)
    jj = jnp.minimum(colp, ROW - 1)
    t = lax.shift_right_logical(jj, 1)
    d = lax.bitwise_and(jj, 1)
    gi_ref[...] = (rows * (M * ROW) + t * (2 * M) + d * 128
                   + lax.shift_right_logical(ids, 7) * 256
                   + lax.bitwise_and(ids, 127))

    asel = jax.lax.dot_general(
        jnp.where(onehot, 1.0, 0.0), anc_ref[...],
        dimension_numbers=(((1,), (0,)), ((), ())),
        preferred_element_type=jnp.float32)
    tm_ref[...] = tgt_ref[...] - asel


def _sc_body(gi_hbm, tm_hbm, msk_hbm, reg_hbm, cls_hbm, out_hbm,
             gi_v, reg_v, tm_v, msk_v, part_s, part_v, sum_v, cls_v, out_v,
             sem_r):
    sid = lax.axis_index("s")
    cid = lax.axis_index("c")
    wid = sid * NC + cid
    base = wid * BPW
    pltpu.sync_copy(gi_hbm.at[pl.ds(base * PAD, BPW * PAD)], gi_v)
    cp_r = pltpu.async_copy(reg_hbm.at[gi_v], reg_v, sem_r)
    pltpu.sync_copy(tm_hbm.at[pl.ds(base, BPW)], tm_v)
    pltpu.sync_copy(msk_hbm.at[pl.ds(base, BPW)], msk_v)
    cp_r.wait()

    def row_body(i, acc):
        for c in range(7):
            off = c * LANES
            r = reg_v[pl.ds(i * PAD + off, LANES)]
            tm = tm_v[i, pl.ds(off, LANES)]
            m = msk_v[i, pl.ds(off, LANES)]
            diff = r - tm
            ad = jnp.abs(diff)
            sl1 = jnp.where(ad < 1.0, 0.5 * diff * diff, ad - 0.5)
            acc = acc + jnp.where(m != 0.0, sl1, 0.0)
        return acc

    acc = lax.fori_loop(0, BPW, row_body, jnp.zeros((LANES,), jnp.float32))
    sum_v[...] = acc
    pltpu.sync_copy(sum_v, part_s.at[sid])
    plsc.subcore_barrier()

    @pl.when(sid == 0)
    def _():
        pltpu.sync_copy(part_s, part_v)
        pltpu.sync_copy(cls_hbm, cls_v)

        def fold(i, acc):
            return acc + part_v[i, :]

        tot = lax.fori_loop(0, NS, fold, jnp.zeros((LANES,), jnp.float32))
        out_v[...] = cls_v[...] * (1.0 / (NC * LANES)) + tot * _INV_N
        pltpu.sync_copy(out_v, out_hbm.at[cid])


@jax.jit
def kernel(cls_out, reg_out, targets, anchors):
    e = targets[0, :, -1, :]
    a_t = anchors[:, -1, :].T
    tgt2p = jnp.pad(targets[0].reshape(N, ROW), ((0, 0), (0, PAD - ROW)))
    msk2p = jnp.pad(targets[1].reshape(N, ROW), ((0, 0), (0, PAD - ROW)))
    anc2p = jnp.pad(anchors.reshape(M, ROW), ((0, 0), (0, PAD - ROW)))

    gi, tm, closs = pl.pallas_call(
        _prep_body,
        out_shape=[
            jax.ShapeDtypeStruct((N, PAD), jnp.int32),
            jax.ShapeDtypeStruct((N, PAD), jnp.float32),
            jax.ShapeDtypeStruct((1, 1), jnp.float32),
        ],
        out_specs=[
            pl.BlockSpec(memory_space=pltpu.VMEM),
            pl.BlockSpec(memory_space=pltpu.VMEM),
            pl.BlockSpec(memory_space=pltpu.SMEM),
        ],
    )(cls_out, e, a_t, tgt2p, anc2p)

    regflat = (reg_out.reshape(N, 4, 128, T, D)
               .transpose(0, 3, 1, 4, 2).reshape(NW))
    cls16 = jnp.broadcast_to(closs.reshape(1), (LANES,))

    mesh = plsc.VectorSubcoreMesh(
        core_axis_name="c", subcore_axis_name="s", num_cores=NC)
    sc = pl.kernel(
        _sc_body,
        out_type=jax.ShapeDtypeStruct((NC, LANES), jnp.float32),
        mesh=mesh,
        compiler_params=pltpu.CompilerParams(
            use_tc_tiling_on_sc=False, needs_layout_passes=False),
        scratch_types=[
            pltpu.VMEM((BPW * PAD,), jnp.int32),
            pltpu.VMEM((BPW * PAD,), jnp.float32),
            pltpu.VMEM((BPW, PAD), jnp.float32),
            pltpu.VMEM((BPW, PAD), jnp.float32),
            pltpu.VMEM_SHARED((NS, LANES), jnp.float32),
            pltpu.VMEM((NS, LANES), jnp.float32),
            pltpu.VMEM((LANES,), jnp.float32),
            pltpu.VMEM((LANES,), jnp.float32),
            pltpu.VMEM((LANES,), jnp.float32),
            pltpu.SemaphoreType.DMA,
        ],
    )
    out16 = sc(gi.reshape(N * PAD), tm, msk2p, regflat, cls16)
    return jnp.sum(out16)


# speedup vs baseline: 956.2931x; 1.0785x over previous
import functools

import jax
import jax.numpy as jnp
from jax import lax
from jax.experimental import pallas as pl
from jax.experimental.pallas import tpu as pltpu
from jax.experimental.pallas import tpu_sc as plsc

N, M, T, D = 1024, 512, 50, 2
ROW = T * D
PAD = 128
NW = N * M * T * D
NC = 2
NS = 16
NWK = NC * NS
BPW = N // NWK
LANES = 16
_INV_N = 1.0 / N


def _prep_body(cls_ref, tgt_ref, anc_ref, gi_ref, tm_ref, closs_ref):
    e0 = tgt_ref[0:N, 98:99]
    e1 = tgt_ref[0:N, 99:100]
    a0 = jnp.transpose(anc_ref[:, 98:99])
    a1 = jnp.transpose(anc_ref[:, 99:100])
    d0 = e0 - a0
    d1 = e1 - a1
    dists = d0 * d0 + d1 * d1
    mind = jnp.min(dists, axis=1, keepdims=True)
    col = lax.broadcasted_iota(jnp.int32, (N, M), 1)
    ids = jnp.min(jnp.where(dists <= mind, col, M), axis=1, keepdims=True)

    cls = cls_ref[...]
    onehot = col == ids
    sel = jnp.sum(jnp.where(onehot, cls, 0.0), axis=1, keepdims=True)
    mx = jnp.max(cls, axis=1, keepdims=True)
    lse = jnp.log(jnp.sum(jnp.exp(cls - mx), axis=1, keepdims=True)) + mx
    closs_ref[0, 0] = jnp.sum(lse - sel) * _INV_N

    rows = lax.broadcasted_iota(jnp.int32, (N, 1), 0)
    colp = lax.broadcasted_iota(jnp.int32, (N, PAD), 1)
    jj = jnp.minimum(colp, ROW - 1)
    t = lax.shift_right_logical(jj, 1)
    d = lax.bitwise_and(jj, 1)
    gi_ref[...] = (rows * (M * ROW) + t * (2 * M) + d * 128
                   + lax.shift_right_logical(ids, 7) * 256
                   + lax.bitwise_and(ids, 127))

    asel = jax.lax.dot_general(
        jnp.where(onehot, 1.0, 0.0), anc_ref[...],
        dimension_numbers=(((1,), (0,)), ((), ())),
        preferred_element_type=jnp.float32)
    tm_ref[...] = tgt_ref[0:N, :] - asel


def _sc_body(gi_hbm, tm_hbm, msk_hbm, reg_hbm, out_hbm,
             gi_v, reg_v, tm_v, msk_v, part_s, part_v, sum_v, out_v,
             sem_r):
    sid = lax.axis_index("s")
    cid = lax.axis_index("c")
    wid = sid * NC + cid
    base = wid * BPW
    pltpu.sync_copy(gi_hbm.at[pl.ds(base * PAD, BPW * PAD)], gi_v)
    cp_r = pltpu.async_copy(reg_hbm.at[gi_v], reg_v, sem_r)
    pltpu.sync_copy(tm_hbm.at[pl.ds(base, BPW)], tm_v)
    pltpu.sync_copy(msk_hbm.at[pl.ds(N + base, BPW)], msk_v)
    cp_r.wait()

    def row_body(i, acc):
        for c in range(7):
            off = c * LANES
            r = reg_v[pl.ds(i * PAD + off, LANES)]
            tm = tm_v[i, pl.ds(off, LANES)]
            m = msk_v[i, pl.ds(off, LANES)]
            diff = r - tm
            ad = jnp.abs(diff)
            sl1 = jnp.where(ad < 1.0, 0.5 * diff * diff, ad - 0.5)
            acc = acc + jnp.where(m != 0.0, sl1, 0.0)
        return acc

    acc = lax.fori_loop(0, BPW, row_body, jnp.zeros((LANES,), jnp.float32))
    sum_v[...] = acc
    pltpu.sync_copy(sum_v, part_s.at[sid])
    plsc.subcore_barrier()

    @pl.when(sid == 0)
    def _():
        pltpu.sync_copy(part_s, part_v)

        def fold(i, acc):
            return acc + part_v[i, :]

        tot = lax.fori_loop(0, NS, fold, jnp.zeros((LANES,), jnp.float32))
        out_v[...] = tot * _INV_N
        pltpu.sync_copy(out_v, out_hbm.at[cid])


@jax.jit
def kernel(cls_out, reg_out, targets, anchors):
    tmk = jnp.pad(targets.reshape(2 * N, ROW), ((0, 0), (0, PAD - ROW)))
    anc2p = jnp.pad(anchors.reshape(M, ROW), ((0, 0), (0, PAD - ROW)))

    gi, tm, closs = pl.pallas_call(
        _prep_body,
        out_shape=[
            jax.ShapeDtypeStruct((N, PAD), jnp.int32),
            jax.ShapeDtypeStruct((N, PAD), jnp.float32),
            jax.ShapeDtypeStruct((1, 1), jnp.float32),
        ],
        out_specs=[
            pl.BlockSpec(memory_space=pltpu.VMEM),
            pl.BlockSpec(memory_space=pltpu.VMEM),
            pl.BlockSpec(memory_space=pltpu.SMEM),
        ],
    )(cls_out, tmk, anc2p)

    regflat = (reg_out.reshape(N, 4, 128, T, D)
               .transpose(0, 3, 1, 4, 2).reshape(NW))

    mesh = plsc.VectorSubcoreMesh(
        core_axis_name="c", subcore_axis_name="s", num_cores=NC)
    sc = pl.kernel(
        _sc_body,
        out_type=jax.ShapeDtypeStruct((NC, LANES), jnp.float32),
        mesh=mesh,
        compiler_params=pltpu.CompilerParams(
            use_tc_tiling_on_sc=False, needs_layout_passes=False),
        scratch_types=[
            pltpu.VMEM((BPW * PAD,), jnp.int32),
            pltpu.VMEM((BPW * PAD,), jnp.float32),
            pltpu.VMEM((BPW, PAD), jnp.float32),
            pltpu.VMEM((BPW, PAD), jnp.float32),
            pltpu.VMEM_SHARED((NS, LANES), jnp.float32),
            pltpu.VMEM((NS, LANES), jnp.float32),
            pltpu.VMEM((LANES,), jnp.float32),
            pltpu.VMEM((LANES,), jnp.float32),
            pltpu.SemaphoreType.DMA,
        ],
    )
    out2 = sc(gi.reshape(N * PAD), tm, tmk, regflat)
    return jnp.sum(out2) + closs[0, 0]


# speedup vs baseline: 957.4814x; 1.0012x over previous
import functools

import jax
import jax.numpy as jnp
from jax import lax
from jax.experimental import pallas as pl
from jax.experimental.pallas import tpu as pltpu
from jax.experimental.pallas import tpu_sc as plsc

N, M, T, D = 1024, 512, 50, 2
ROW = T * D
PAD = 128
NW = N * M * T * D
NC = 2
NS = 16
NWK = NC * NS
BPW = N // NWK
LANES = 16
_INV_N = 1.0 / N


def _prep_body(cls_ref, tgt_ref, anc_ref, gi_ref, tm_ref, closs_ref):
    e0 = tgt_ref[0:N, 98:99]
    e1 = tgt_ref[0:N, 99:100]
    a0 = jnp.transpose(anc_ref[:, 98:99])
    a1 = jnp.transpose(anc_ref[:, 99:100])
    d0 = e0 - a0
    d1 = e1 - a1
    dists = d0 * d0 + d1 * d1
    mind = jnp.min(dists, axis=1, keepdims=True)
    col = lax.broadcasted_iota(jnp.int32, (N, M), 1)
    ids = jnp.min(jnp.where(dists <= mind, col, M), axis=1, keepdims=True)

    cls = cls_ref[...]
    onehot = col == ids
    sel = jnp.sum(jnp.where(onehot, cls, 0.0), axis=1, keepdims=True)
    mx = jnp.max(cls, axis=1, keepdims=True)
    lse = jnp.log(jnp.sum(jnp.exp(cls - mx), axis=1, keepdims=True)) + mx
    closs_ref[0, 0] = jnp.sum(lse - sel) * _INV_N

    rows = lax.broadcasted_iota(jnp.int32, (N, 1), 0)
    colp = lax.broadcasted_iota(jnp.int32, (N, PAD), 1)
    jj = jnp.minimum(colp, ROW - 1)
    t = lax.shift_right_logical(jj, 1)
    d = lax.bitwise_and(jj, 1)
    gi_ref[...] = (rows * (M * ROW) + t * (2 * M) + d * 128
                   + lax.shift_right_logical(ids, 7) * 256
                   + lax.bitwise_and(ids, 127))

    asel = jax.lax.dot_general(
        jnp.where(onehot, 1.0, 0.0), anc_ref[...],
        dimension_numbers=(((1,), (0,)), ((), ())),
        preferred_element_type=jnp.float32)
    tm_ref[...] = tgt_ref[0:N, :] - asel


def _sc_body(gi_hbm, tm_hbm, msk_hbm, reg_hbm, out_hbm,
             gi_v, reg_v, tm_v, msk_v, part_s, part_v, sum_v, out_v,
             sem_a, sem_b):
    sid = lax.axis_index("s")
    cid = lax.axis_index("c")
    wid = sid * NC + cid
    base = wid * BPW
    half = BPW // 2 * PAD
    pltpu.sync_copy(gi_hbm.at[pl.ds(base * PAD, BPW * PAD)], gi_v)
    cp_a = pltpu.async_copy(reg_hbm.at[gi_v.at[pl.ds(0, half)]],
                            reg_v.at[pl.ds(0, half)], sem_a)
    cp_b = pltpu.async_copy(reg_hbm.at[gi_v.at[pl.ds(half, half)]],
                            reg_v.at[pl.ds(half, half)], sem_b)
    pltpu.sync_copy(tm_hbm.at[pl.ds(base, BPW)], tm_v)
    pltpu.sync_copy(msk_hbm.at[pl.ds(N + base, BPW)], msk_v)

    def row_body(i, acc):
        for c in range(7):
            off = c * LANES
            r = reg_v[pl.ds(i * PAD + off, LANES)]
            tm = tm_v[i, pl.ds(off, LANES)]
            m = msk_v[i, pl.ds(off, LANES)]
            diff = r - tm
            ad = jnp.abs(diff)
            sl1 = jnp.where(ad < 1.0, 0.5 * diff * diff, ad - 0.5)
            acc = acc + jnp.where(m != 0.0, sl1, 0.0)
        return acc

    cp_a.wait()
    acc = lax.fori_loop(0, BPW // 2, row_body,
                        jnp.zeros((LANES,), jnp.float32))
    cp_b.wait()
    acc = lax.fori_loop(BPW // 2, BPW, row_body, acc)
    sum_v[...] = acc
    pltpu.sync_copy(sum_v, part_s.at[sid])
    plsc.subcore_barrier()

    @pl.when(sid == 0)
    def _():
        pltpu.sync_copy(part_s, part_v)

        def fold(i, acc):
            return acc + part_v[i, :]

        tot = lax.fori_loop(0, NS, fold, jnp.zeros((LANES,), jnp.float32))
        out_v[...] = tot * _INV_N
        pltpu.sync_copy(out_v, out_hbm.at[cid])


@jax.jit
def kernel(cls_out, reg_out, targets, anchors):
    tmk = jnp.pad(targets.reshape(2 * N, ROW), ((0, 0), (0, PAD - ROW)))
    anc2p = jnp.pad(anchors.reshape(M, ROW), ((0, 0), (0, PAD - ROW)))

    gi, tm, closs = pl.pallas_call(
        _prep_body,
        out_shape=[
            jax.ShapeDtypeStruct((N, PAD), jnp.int32),
            jax.ShapeDtypeStruct((N, PAD), jnp.float32),
            jax.ShapeDtypeStruct((1, 1), jnp.float32),
        ],
        out_specs=[
            pl.BlockSpec(memory_space=pltpu.VMEM),
            pl.BlockSpec(memory_space=pltpu.VMEM),
            pl.BlockSpec(memory_space=pltpu.SMEM),
        ],
    )(cls_out, tmk, anc2p)

    regflat = (reg_out.reshape(N, 4, 128, T, D)
               .transpose(0, 3, 1, 4, 2).reshape(NW))

    mesh = plsc.VectorSubcoreMesh(
        core_axis_name="c", subcore_axis_name="s", num_cores=NC)
    sc = pl.kernel(
        _sc_body,
        out_type=jax.ShapeDtypeStruct((NC, LANES), jnp.float32),
        mesh=mesh,
        compiler_params=pltpu.CompilerParams(
            use_tc_tiling_on_sc=False, needs_layout_passes=False),
        scratch_types=[
            pltpu.VMEM((BPW * PAD,), jnp.int32),
            pltpu.VMEM((BPW * PAD,), jnp.float32),
            pltpu.VMEM((BPW, PAD), jnp.float32),
            pltpu.VMEM((BPW, PAD), jnp.float32),
            pltpu.VMEM_SHARED((NS, LANES), jnp.float32),
            pltpu.VMEM((NS, LANES), jnp.float32),
            pltpu.VMEM((LANES,), jnp.float32),
            pltpu.VMEM((LANES,), jnp.float32),
            pltpu.SemaphoreType.DMA,
            pltpu.SemaphoreType.DMA,
        ],
    )
    out2 = sc(gi.reshape(N * PAD), tm, tmk, regflat)
    return jnp.sum(out2) + closs[0, 0]


# speedup vs baseline: 997.6622x; 1.0420x over previous
import functools

import jax
import jax.numpy as jnp
from jax import lax
from jax.experimental import pallas as pl
from jax.experimental.pallas import tpu as pltpu
from jax.experimental.pallas import tpu_sc as plsc

N, M, T, D = 1024, 512, 50, 2
ROW = T * D
PAD = 128
NWORDS = N * M * T * D
NC = 2
NS = 16
NWK = NC * NS
BPW = N // NWK
LANES = 16
_INV_N = 1.0 / N


def _argmin_body(tgt_ref, anc_ref, ids_ref, gi_ref):
    e0 = tgt_ref[0:N, 98:99]
    e1 = tgt_ref[0:N, 99:100]
    a0 = jnp.transpose(anc_ref[:, 98:99])
    a1 = jnp.transpose(anc_ref[:, 99:100])
    d0 = e0 - a0
    d1 = e1 - a1
    dists = d0 * d0 + d1 * d1
    mind = jnp.min(dists, axis=1, keepdims=True)
    col = lax.broadcasted_iota(jnp.int32, (N, M), 1)
    ids = jnp.min(jnp.where(dists <= mind, col, M), axis=1, keepdims=True)
    ids_ref[...] = ids

    rows = lax.broadcasted_iota(jnp.int32, (N, 1), 0)
    colp = lax.broadcasted_iota(jnp.int32, (N, PAD), 1)
    jj = jnp.minimum(colp, ROW - 1)
    t = lax.shift_right_logical(jj, 1)
    d = lax.bitwise_and(jj, 1)
    gi_ref[...] = (rows * (M * ROW) + t * (2 * M) + d * 128
                   + lax.shift_right_logical(ids, 7) * 256
                   + lax.bitwise_and(ids, 127))


def _cls_tm_body(cls_ref, ids_ref, tgt_ref, anc_ref, tm_ref, closs_ref):
    ids = ids_ref[...]
    col = lax.broadcasted_iota(jnp.int32, (N, M), 1)
    onehot = col == ids
    cls = cls_ref[...]
    sel = jnp.sum(jnp.where(onehot, cls, 0.0), axis=1, keepdims=True)
    mx = jnp.max(cls, axis=1, keepdims=True)
    lse = jnp.log(jnp.sum(jnp.exp(cls - mx), axis=1, keepdims=True)) + mx
    closs_ref[0, 0] = jnp.sum(lse - sel) * _INV_N

    asel = jax.lax.dot_general(
        jnp.where(onehot, 1.0, 0.0), anc_ref[...],
        dimension_numbers=(((1,), (0,)), ((), ())),
        preferred_element_type=jnp.float32)
    tm_ref[...] = tgt_ref[0:N, :] - asel


def _sc_body(gi_hbm, reg_hbm, out_hbm, gi_v, reg_v, sem_a, sem_b):
    sid = lax.axis_index("s")
    cid = lax.axis_index("c")
    wid = sid * NC + cid
    base = wid * BPW * PAD
    half = BPW // 2 * PAD
    pltpu.sync_copy(gi_hbm.at[pl.ds(base, BPW * PAD)], gi_v)
    cp_a = pltpu.async_copy(reg_hbm.at[gi_v.at[pl.ds(0, half)]],
                            reg_v.at[pl.ds(0, half)], sem_a)
    cp_b = pltpu.async_copy(reg_hbm.at[gi_v.at[pl.ds(half, half)]],
                            reg_v.at[pl.ds(half, half)], sem_b)
    cp_a.wait()
    pltpu.sync_copy(reg_v.at[pl.ds(0, half)], out_hbm.at[pl.ds(base, half)])
    cp_b.wait()
    pltpu.sync_copy(reg_v.at[pl.ds(half, half)],
                    out_hbm.at[pl.ds(base + half, half)])


def _loss_body(g_ref, tm_ref, tgt_ref, closs_ref, out_ref):
    diff = g_ref[...] - tm_ref[...]
    ad = jnp.abs(diff)
    sl1 = jnp.where(ad < 1.0, 0.5 * diff * diff, ad - 0.5)
    msk = tgt_ref[N:2 * N, :]
    val = jnp.where(msk != 0.0, sl1, 0.0)
    out_ref[0, 0] = jnp.sum(val) * _INV_N + closs_ref[0, 0]


@jax.jit
def kernel(cls_out, reg_out, targets, anchors):
    tmk = jnp.pad(targets.reshape(2 * N, ROW), ((0, 0), (0, PAD - ROW)))
    anc2p = jnp.pad(anchors.reshape(M, ROW), ((0, 0), (0, PAD - ROW)))

    ids, gi = pl.pallas_call(
        _argmin_body,
        out_shape=[
            jax.ShapeDtypeStruct((N, 1), jnp.int32),
            jax.ShapeDtypeStruct((N, PAD), jnp.int32),
        ],
    )(tmk, anc2p)

    regflat = (reg_out.reshape(N, 4, 128, T, D)
               .transpose(0, 3, 1, 4, 2).reshape(NWORDS))

    mesh = plsc.VectorSubcoreMesh(
        core_axis_name="c", subcore_axis_name="s", num_cores=NC)
    sc = pl.kernel(
        _sc_body,
        out_type=jax.ShapeDtypeStruct((N * PAD,), jnp.float32),
        mesh=mesh,
        compiler_params=pltpu.CompilerParams(
            use_tc_tiling_on_sc=False, needs_layout_passes=False),
        scratch_types=[
            pltpu.VMEM((BPW * PAD,), jnp.int32),
            pltpu.VMEM((BPW * PAD,), jnp.float32),
            pltpu.SemaphoreType.DMA,
            pltpu.SemaphoreType.DMA,
        ],
    )
    gathered = sc(gi.reshape(N * PAD), regflat)

    tm, closs = pl.pallas_call(
        _cls_tm_body,
        out_shape=[
            jax.ShapeDtypeStruct((N, PAD), jnp.float32),
            jax.ShapeDtypeStruct((1, 1), jnp.float32),
        ],
        out_specs=[
            pl.BlockSpec(memory_space=pltpu.VMEM),
            pl.BlockSpec(memory_space=pltpu.SMEM),
        ],
    )(cls_out, ids, tmk, anc2p)

    out = pl.pallas_call(
        _loss_body,
        in_specs=[
            pl.BlockSpec(memory_space=pltpu.VMEM),
            pl.BlockSpec(memory_space=pltpu.VMEM),
            pl.BlockSpec(memory_space=pltpu.VMEM),
            pl.BlockSpec(memory_space=pltpu.SMEM),
        ],
        out_shape=jax.ShapeDtypeStruct((1, 1), jnp.float32),
        out_specs=pl.BlockSpec(memory_space=pltpu.SMEM),
    )(gathered.reshape(N, PAD), tm, tmk, closs)
    return out[0, 0]


# speedup vs baseline: 998.4318x; 1.0008x over previous
import functools

import jax
import jax.numpy as jnp
from jax import lax
from jax.experimental import pallas as pl
from jax.experimental.pallas import tpu as pltpu
from jax.experimental.pallas import tpu_sc as plsc

N, M, T, D = 1024, 512, 50, 2
ROW = T * D
PAD = 128
NWORDS = N * M * T * D
NC = 2
NS = 16
NWK = NC * NS
BPW = N // NWK
LANES = 16
_INV_N = 1.0 / N


def _argmin_body(tgt_ref, anc_ref, ids_ref, gi_ref):
    e0 = tgt_ref[0:N, 98:99]
    e1 = tgt_ref[0:N, 99:100]
    a0 = jnp.transpose(anc_ref[:, 98:99])
    a1 = jnp.transpose(anc_ref[:, 99:100])
    d0 = e0 - a0
    d1 = e1 - a1
    dists = d0 * d0 + d1 * d1
    mind = jnp.min(dists, axis=1, keepdims=True)
    col = lax.broadcasted_iota(jnp.int32, (N, M), 1)
    ids = jnp.min(jnp.where(dists <= mind, col, M), axis=1, keepdims=True)
    ids_ref[...] = ids

    rows = lax.broadcasted_iota(jnp.int32, (N, 1), 0)
    colp = lax.broadcasted_iota(jnp.int32, (N, PAD), 1)
    jj = jnp.minimum(colp, ROW - 1)
    t = lax.shift_right_logical(jj, 1)
    d = lax.bitwise_and(jj, 1)
    gi_ref[...] = (rows * (M * ROW) + t * (2 * M) + d * 128
                   + lax.shift_right_logical(ids, 7) * 256
                   + lax.bitwise_and(ids, 127))


def _cls_tm_body(cls_ref, ids_ref, tgt_ref, anc_ref, tm_ref, closs_ref):
    ids = ids_ref[...]
    col = lax.broadcasted_iota(jnp.int32, (N, M), 1)
    onehot = col == ids
    cls = cls_ref[...]
    sel = jnp.sum(jnp.where(onehot, cls, 0.0), axis=1, keepdims=True)
    mx = jnp.max(cls, axis=1, keepdims=True)
    lse = jnp.log(jnp.sum(jnp.exp(cls - mx), axis=1, keepdims=True)) + mx
    closs_ref[0, 0] = jnp.sum(lse - sel) * _INV_N

    asel = jax.lax.dot_general(
        jnp.where(onehot, 1.0, 0.0), anc_ref[...],
        dimension_numbers=(((1,), (0,)), ((), ())),
        preferred_element_type=jnp.float32)
    tm_ref[...] = tgt_ref[0:N, :] - asel


def _loss_body(g_ref, tm_ref, tgt_ref, closs_ref, out_ref):
    diff = g_ref[:, 0:ROW] - tm_ref[...]
    ad = jnp.abs(diff)
    sl1 = jnp.where(ad < 1.0, 0.5 * diff * diff, ad - 0.5)
    msk = tgt_ref[N:2 * N, :]
    val = jnp.where(msk != 0.0, sl1, 0.0)
    out_ref[0, 0] = jnp.sum(val) * _INV_N + closs_ref[0, 0]


def _sc_body(gi_hbm, reg_hbm, out_hbm, gi_v, reg_v, sem_a, sem_b):
    sid = lax.axis_index("s")
    cid = lax.axis_index("c")
    wid = sid * NC + cid
    base = wid * BPW * PAD
    half = BPW // 2 * PAD
    pltpu.sync_copy(gi_hbm.at[pl.ds(base, BPW * PAD)], gi_v)
    cp_a = pltpu.async_copy(reg_hbm.at[gi_v.at[pl.ds(0, half)]],
                            reg_v.at[pl.ds(0, half)], sem_a)
    cp_b = pltpu.async_copy(reg_hbm.at[gi_v.at[pl.ds(half, half)]],
                            reg_v.at[pl.ds(half, half)], sem_b)
    cp_a.wait()
    pltpu.sync_copy(reg_v.at[pl.ds(0, half)], out_hbm.at[pl.ds(base, half)])
    cp_b.wait()
    pltpu.sync_copy(reg_v.at[pl.ds(half, half)],
                    out_hbm.at[pl.ds(base + half, half)])


@jax.jit
def kernel(cls_out, reg_out, targets, anchors):
    tmk = targets.reshape(2 * N, ROW)
    anc2p = anchors.reshape(M, ROW)

    ids, gi = pl.pallas_call(
        _argmin_body,
        out_shape=[
            jax.ShapeDtypeStruct((N, 1), jnp.int32),
            jax.ShapeDtypeStruct((N, PAD), jnp.int32),
        ],
    )(tmk, anc2p)

    regflat = (reg_out.reshape(N, 4, 128, T, D)
               .transpose(0, 3, 1, 4, 2).reshape(NWORDS))

    mesh = plsc.VectorSubcoreMesh(
        core_axis_name="c", subcore_axis_name="s", num_cores=NC)
    sc = pl.kernel(
        _sc_body,
        out_type=jax.ShapeDtypeStruct((N * PAD,), jnp.float32),
        mesh=mesh,
        compiler_params=pltpu.CompilerParams(
            use_tc_tiling_on_sc=False, needs_layout_passes=False),
        scratch_types=[
            pltpu.VMEM((BPW * PAD,), jnp.int32),
            pltpu.VMEM((BPW * PAD,), jnp.float32),
            pltpu.SemaphoreType.DMA,
            pltpu.SemaphoreType.DMA,
        ],
    )
    gathered = sc(gi.reshape(N * PAD), regflat)

    tm, closs = pl.pallas_call(
        _cls_tm_body,
        out_shape=[
            jax.ShapeDtypeStruct((N, ROW), jnp.float32),
            jax.ShapeDtypeStruct((1, 1), jnp.float32),
        ],
        out_specs=[
            pl.BlockSpec(memory_space=pltpu.VMEM),
            pl.BlockSpec(memory_space=pltpu.SMEM),
        ],
    )(cls_out, ids, tmk, anc2p)

    out = pl.pallas_call(
        _loss_body,
        in_specs=[
            pl.BlockSpec(memory_space=pltpu.VMEM),
            pl.BlockSpec(memory_space=pltpu.VMEM),
            pl.BlockSpec(memory_space=pltpu.VMEM),
            pl.BlockSpec(memory_space=pltpu.SMEM),
        ],
        out_shape=jax.ShapeDtypeStruct((1, 1), jnp.float32),
        out_specs=pl.BlockSpec(memory_space=pltpu.SMEM),
    )(gathered.reshape(N, PAD), tm, tmk, closs)
    return out[0, 0]


# speedup vs baseline: 1001.0227x; 1.0026x over previous
import jax
import jax.numpy as jnp
from jax import lax
from jax.experimental import pallas as pl
from jax.experimental.pallas import tpu as pltpu
from jax.experimental.pallas import tpu_sc as plsc

N, M, T, D = 1024, 512, 50, 2
ROW = T * D
PAD = 128
NWORDS = N * M * T * D
NC = 2
NS = 16
NWK = NC * NS
BPW = N // NWK
LANES = 16
_INV_N = 1.0 / N


def _argmin_body(tgt_ref, anc_ref, ids_ref, gi_ref):
    e0 = tgt_ref[0:N, 98:99]
    e1 = tgt_ref[0:N, 99:100]
    a0 = jnp.transpose(anc_ref[:, 98:99])
    a1 = jnp.transpose(anc_ref[:, 99:100])
    d0 = e0 - a0
    d1 = e1 - a1
    dists = d0 * d0 + d1 * d1
    mind = jnp.min(dists, axis=1, keepdims=True)
    col = lax.broadcasted_iota(jnp.int32, (N, M), 1)
    ids = jnp.min(jnp.where(dists <= mind, col, M), axis=1, keepdims=True)
    ids_ref[...] = ids

    rows = lax.broadcasted_iota(jnp.int32, (N, 1), 0)
    colp = lax.broadcasted_iota(jnp.int32, (N, PAD), 1)
    jj = jnp.minimum(colp, ROW - 1)
    t = lax.shift_right_logical(jj, 1)
    d = lax.bitwise_and(jj, 1)
    gi_ref[...] = (rows * (M * ROW) + t * (2 * M) + d * 128
                   + lax.shift_right_logical(ids, 7) * 256
                   + lax.bitwise_and(ids, 127))


def _cls_tm_body(cls_ref, ids_ref, tgt_ref, anc_ref, tm_ref, closs_ref):
    ids = ids_ref[...]
    col = lax.broadcasted_iota(jnp.int32, (N, M), 1)
    onehot = col == ids
    cls = cls_ref[...]
    sel = jnp.sum(jnp.where(onehot, cls, 0.0), axis=1, keepdims=True)
    mx = jnp.max(cls, axis=1, keepdims=True)
    lse = jnp.log(jnp.sum(jnp.exp(cls - mx), axis=1, keepdims=True)) + mx
    closs_ref[0, 0] = jnp.sum(lse - sel) * _INV_N

    asel = jax.lax.dot_general(
        jnp.where(onehot, 1.0, 0.0), anc_ref[...],
        dimension_numbers=(((1,), (0,)), ((), ())),
        preferred_element_type=jnp.float32)
    tm_ref[...] = tgt_ref[0:N, :] - asel


def _loss_body(g_ref, tm_ref, tgt_ref, closs_ref, out_ref):
    diff = g_ref[:, 0:ROW] - tm_ref[...]
    ad = jnp.abs(diff)
    sl1 = jnp.where(ad < 1.0, 0.5 * diff * diff, ad - 0.5)
    msk = tgt_ref[N:2 * N, :]
    val = jnp.where(msk != 0.0, sl1, 0.0)
    out_ref[0, 0] = jnp.sum(val) * _INV_N + closs_ref[0, 0]


def _sc_body(gi_hbm, reg_hbm, out_hbm, gi_v, reg_v, sem_a, sem_b):
    sid = lax.axis_index("s")
    cid = lax.axis_index("c")
    wid = sid * NC + cid
    base = wid * BPW * PAD
    half = BPW // 2 * PAD
    pltpu.sync_copy(gi_hbm.at[pl.ds(base, BPW * PAD)], gi_v)
    cp_a = pltpu.async_copy(reg_hbm.at[gi_v.at[pl.ds(0, half)]],
                            reg_v.at[pl.ds(0, half)], sem_a)
    cp_b = pltpu.async_copy(reg_hbm.at[gi_v.at[pl.ds(half, half)]],
                            reg_v.at[pl.ds(half, half)], sem_b)
    cp_a.wait()
    pltpu.sync_copy(reg_v.at[pl.ds(0, half)], out_hbm.at[pl.ds(base, half)])
    cp_b.wait()
    pltpu.sync_copy(reg_v.at[pl.ds(half, half)],
                    out_hbm.at[pl.ds(base + half, half)])


@jax.jit
def kernel(cls_out, reg_out, targets, anchors):
    tmk = targets.reshape(2 * N, ROW)
    anc2p = anchors.reshape(M, ROW)

    ids, gi = pl.pallas_call(
        _argmin_body,
        out_shape=[
            jax.ShapeDtypeStruct((N, 1), jnp.int32),
            jax.ShapeDtypeStruct((N, PAD), jnp.int32),
        ],
    )(tmk, anc2p)

    regflat = (reg_out.reshape(N, 4, 128, T, D)
               .transpose(0, 3, 1, 4, 2).reshape(NWORDS))

    mesh = plsc.VectorSubcoreMesh(
        core_axis_name="c", subcore_axis_name="s", num_cores=NC)
    sc = pl.kernel(
        _sc_body,
        out_type=jax.ShapeDtypeStruct((N * PAD,), jnp.float32),
        mesh=mesh,
        compiler_params=pltpu.CompilerParams(
            use_tc_tiling_on_sc=False, needs_layout_passes=False),
        scratch_types=[
            pltpu.VMEM((BPW * PAD,), jnp.int32),
            pltpu.VMEM((BPW * PAD,), jnp.float32),
            pltpu.SemaphoreType.DMA,
            pltpu.SemaphoreType.DMA,
        ],
    )
    gathered = sc(gi.reshape(N * PAD), regflat)

    tm, closs = pl.pallas_call(
        _cls_tm_body,
        out_shape=[
            jax.ShapeDtypeStruct((N, ROW), jnp.float32),
            jax.ShapeDtypeStruct((1, 1), jnp.float32),
        ],
        out_specs=[
            pl.BlockSpec(memory_space=pltpu.VMEM),
            pl.BlockSpec(memory_space=pltpu.SMEM),
        ],
    )(cls_out, ids, tmk, anc2p)

    out = pl.pallas_call(
        _loss_body,
        in_specs=[
            pl.BlockSpec(memory_space=pltpu.VMEM),
            pl.BlockSpec(memory_space=pltpu.VMEM),
            pl.BlockSpec(memory_space=pltpu.VMEM),
            pl.BlockSpec(memory_space=pltpu.SMEM),
        ],
        out_shape=jax.ShapeDtypeStruct((1, 1), jnp.float32),
        out_specs=pl.BlockSpec(memory_space=pltpu.SMEM),
    )(gathered.reshape(N, PAD), tm, tmk, closs)
    return out[0, 0]
